# Initial kernel scaffold; baseline (speedup 1.0000x reference)
import jax
import jax.numpy as jnp
from jax.experimental import pallas as pl


def kernel(h, x, edge_index, W_m1, b_m1, W_m2, b_m2, W_c1, b_c1, W_c2, W_n1, b_n1, W_n2, b_n2, ln_g, ln_b):
    raise NotImplementedError("write your pallas kernel here")


# speedup vs baseline: 2.8037x; 2.8037x over previous
import functools

import jax
import jax.numpy as jnp
from jax import lax
from jax.experimental import pallas as pl
from jax.experimental.pallas import tpu as pltpu
from jax.experimental.pallas import tpu_sc as plsc

_F32 = jnp.float32
_HIGH = lax.Precision.HIGHEST
_NC, _NS, _CH = 2, 16, 128


def _silu(v):
    return v * jax.nn.sigmoid(v)


def _dot(a, b):
    return jnp.dot(a, b, preferred_element_type=_F32, precision=_HIGH)


def _sc_gather(hA, hB, xpad, row_p, col_p):
    H = hA.shape[1]
    EP = row_p.shape[0]
    EPW = EP // (_NC * _NS)
    nch = EPW // _CH
    mesh = plsc.VectorSubcoreMesh(core_axis_name="c", subcore_axis_name="s")

    @functools.partial(
        pl.kernel, mesh=mesh,
        compiler_params=pltpu.CompilerParams(use_tc_tiling_on_sc=False),
        out_type=[
            jax.ShapeDtypeStruct((EP, H), _F32),
            jax.ShapeDtypeStruct((EP, H), _F32),
            jax.ShapeDtypeStruct((EP, 16), _F32),
            jax.ShapeDtypeStruct((EP, 16), _F32),
        ],
        scratch_types=[
            pltpu.VMEM((_CH,), jnp.int32),
            pltpu.VMEM((_CH,), jnp.int32),
            pltpu.VMEM((_CH, H), _F32),
            pltpu.VMEM((_CH, H), _F32),
            pltpu.VMEM((_CH, 16), _F32),
            pltpu.VMEM((_CH, 16), _F32),
            pltpu.SemaphoreType.DMA,
        ],
    )
    def gather_k(hA_hbm, hB_hbm, xp_hbm, row_hbm, col_hbm,
                 gA_hbm, gB_hbm, xr_hbm, xc_hbm,
                 ir_v, ic_v, bA, bB, bxr, bxc, sem):
        c = lax.axis_index("c")
        s = lax.axis_index("s")
        base = (s * _NC + c) * EPW

        def body(k, carry):
            off = pl.multiple_of(base + k * _CH, _CH)
            pltpu.sync_copy(row_hbm.at[pl.ds(off, _CH)], ir_v)
            pltpu.sync_copy(col_hbm.at[pl.ds(off, _CH)], ic_v)
            d1 = pltpu.async_copy(hA_hbm.at[ir_v], bA, sem)
            d2 = pltpu.async_copy(hB_hbm.at[ic_v], bB, sem)
            d3 = pltpu.async_copy(xp_hbm.at[ir_v], bxr, sem)
            d4 = pltpu.async_copy(xp_hbm.at[ic_v], bxc, sem)
            d1.wait(); d2.wait(); d3.wait(); d4.wait()
            pltpu.sync_copy(bA, gA_hbm.at[pl.ds(off, _CH)])
            pltpu.sync_copy(bB, gB_hbm.at[pl.ds(off, _CH)])
            pltpu.sync_copy(bxr, xr_hbm.at[pl.ds(off, _CH)])
            pltpu.sync_copy(bxc, xc_hbm.at[pl.ds(off, _CH)])
            return carry

        lax.fori_loop(0, nch, body, 0)

    return gather_k(hA, hB, xpad, row_p, col_p)


def _sc_scatter(msg, crd, col_p, NP):
    H = msg.shape[1]
    EP = col_p.shape[0]
    EPW = EP // (_NC * _NS)
    nch = EPW // _CH
    NPT = NP // _NS
    zeros_big = jnp.zeros((NPT, H), _F32)
    zeros_sm = jnp.zeros((NPT, 16), _F32)
    mesh = plsc.VectorSubcoreMesh(core_axis_name="c", subcore_axis_name="s")

    @functools.partial(
        pl.kernel, mesh=mesh,
        compiler_params=pltpu.CompilerParams(use_tc_tiling_on_sc=False),
        out_type=[
            jax.ShapeDtypeStruct((_NC, NP, H), _F32),
            jax.ShapeDtypeStruct((_NC, NP, 16), _F32),
        ],
        scratch_types=[
            pltpu.VMEM((_CH,), jnp.int32),
            pltpu.VMEM((_CH, H), _F32),
            pltpu.VMEM((_CH, 16), _F32),
            pltpu.VMEM_SHARED((NP, H), _F32),
            pltpu.VMEM_SHARED((NP, 16), _F32),
            pltpu.SemaphoreType.DMA,
        ],
    )
    def scatter_k(msg_hbm, crd_hbm, col_hbm, z128_hbm, z16_hbm,
                  agg_hbm, cacc_hbm, ci_v, mb_v, cb_v, aggs, crds, sem):
        c = lax.axis_index("c")
        s = lax.axis_index("s")
        pltpu.sync_copy(z128_hbm, aggs.at[pl.ds(s * NPT, NPT)])
        pltpu.sync_copy(z16_hbm, crds.at[pl.ds(s * NPT, NPT)])
        plsc.subcore_barrier()
        base = (c * _NS + s) * EPW

        def body(k, carry):
            off = pl.multiple_of(base + k * _CH, _CH)
            pltpu.sync_copy(col_hbm.at[pl.ds(off, _CH)], ci_v)
            d1 = pltpu.async_copy(msg_hbm.at[pl.ds(off, _CH)], mb_v, sem)
            d2 = pltpu.async_copy(crd_hbm.at[pl.ds(off, _CH)], cb_v, sem)
            d1.wait(); d2.wait()
            pltpu.sync_copy(mb_v, aggs.at[ci_v], add=True)
            pltpu.sync_copy(cb_v, crds.at[ci_v], add=True)
            return carry

        lax.fori_loop(0, nch, body, 0)
        plsc.subcore_barrier()
        pltpu.sync_copy(aggs.at[pl.ds(s * NPT, NPT)],
                        agg_hbm.at[c, pl.ds(s * NPT, NPT)])
        pltpu.sync_copy(crds.at[pl.ds(s * NPT, NPT)],
                        cacc_hbm.at[c, pl.ds(s * NPT, NPT)])

    return scatter_k(msg, crd, col_p, zeros_big, zeros_sm)


def kernel(h, x, edge_index, W_m1, b_m1, W_m2, b_m2, W_c1, b_c1, W_c2,
           W_n1, b_n1, W_n2, b_n2, ln_g, ln_b):
    N, H = h.shape
    E = edge_index.shape[1]
    NW = _NC * _NS

    W_m1aT = W_m1[:, :H].T
    W_m1bT = W_m1[:, H:2 * H].T
    w_d = W_m1[:, 2 * H].reshape(1, H)
    W_m2T = W_m2.T
    W_c1T = W_c1.T
    w_c2 = W_c2.reshape(1, H)
    W_n1aT = W_n1[:, :H].T
    W_n1bT = W_n1[:, H:].T
    W_n2T = W_n2.T
    b_m1r = b_m1.reshape(1, H)
    b_m2r = b_m2.reshape(1, H)
    b_c1r = b_c1.reshape(1, H)
    b_n1r = b_n1.reshape(1, H)
    b_n2r = b_n2.reshape(1, H)
    ln_gr = ln_g.reshape(1, H)
    ln_br = ln_b.reshape(1, H)

    xpad = jnp.pad(x, ((0, 0), (0, 16 - x.shape[1])))

    EPW = -(-E // (NW * _CH)) * _CH
    EP = EPW * NW
    row_p = jnp.concatenate([edge_index[0],
                             jnp.zeros((EP - E,), jnp.int32)])
    col_p = jnp.concatenate([edge_index[1],
                             jnp.full((EP - E,), N, jnp.int32)])
    NP = N + 16

    BN = 2000

    def pre_body(h_ref, wa_ref, wb_ref, bm1_ref, hA_ref, hB_ref):
        hv = h_ref[...]
        hA_ref[...] = _dot(hv, wa_ref[...]) + bm1_ref[...]
        hB_ref[...] = _dot(hv, wb_ref[...])

    hA, hB = pl.pallas_call(
        pre_body,
        grid=(N // BN,),
        in_specs=[
            pl.BlockSpec((BN, H), lambda i: (i, 0)),
            pl.BlockSpec((H, H), lambda i: (0, 0)),
            pl.BlockSpec((H, H), lambda i: (0, 0)),
            pl.BlockSpec((1, H), lambda i: (0, 0)),
        ],
        out_specs=[
            pl.BlockSpec((BN, H), lambda i: (i, 0)),
            pl.BlockSpec((BN, H), lambda i: (i, 0)),
        ],
        out_shape=[
            jax.ShapeDtypeStruct((N, H), _F32),
            jax.ShapeDtypeStruct((N, H), _F32),
        ],
        compiler_params=pltpu.CompilerParams(
            dimension_semantics=("parallel",)),
    )(h, W_m1aT, W_m1bT, b_m1r)

    gA, gB, xr, xc = _sc_gather(hA, hB, xpad, row_p, col_p)

    BE = 2048

    def edge_body(gA_ref, gB_ref, xr_ref, xc_ref, wd_ref, wm2_ref, bm2_ref,
                  wc1_ref, bc1_ref, wc2_ref, msg_ref, crd_ref):
        rel = xr_ref[...] - xc_ref[...]
        dist = jnp.sqrt(jnp.sum(rel * rel, axis=1, keepdims=True))
        pre = gA_ref[...] + gB_ref[...] + dist * wd_ref[...]
        m1 = _silu(pre)
        msg = _silu(_dot(m1, wm2_ref[...]) + bm2_ref[...])
        msg_ref[...] = msg
        cc = _silu(_dot(msg, wc1_ref[...]) + bc1_ref[...])
        cm = jnp.tanh(jnp.sum(cc * wc2_ref[...], axis=1, keepdims=True))
        crd = cm * (rel / (dist + 1e-8))
        lane = lax.broadcasted_iota(jnp.int32, crd.shape, 1)
        crd_ref[...] = jnp.where(lane == 3, 1.0, crd)

    msg, crd = pl.pallas_call(
        edge_body,
        grid=(EP // BE,),
        in_specs=[
            pl.BlockSpec((BE, H), lambda i: (i, 0)),
            pl.BlockSpec((BE, H), lambda i: (i, 0)),
            pl.BlockSpec((BE, 16), lambda i: (i, 0)),
            pl.BlockSpec((BE, 16), lambda i: (i, 0)),
            pl.BlockSpec((1, H), lambda i: (0, 0)),
            pl.BlockSpec((H, H), lambda i: (0, 0)),
            pl.BlockSpec((1, H), lambda i: (0, 0)),
            pl.BlockSpec((H, H), lambda i: (0, 0)),
            pl.BlockSpec((1, H), lambda i: (0, 0)),
            pl.BlockSpec((1, H), lambda i: (0, 0)),
        ],
        out_specs=[
            pl.BlockSpec((BE, H), lambda i: (i, 0)),
            pl.BlockSpec((BE, 16), lambda i: (i, 0)),
        ],
        out_shape=[
            jax.ShapeDtypeStruct((EP, H), _F32),
            jax.ShapeDtypeStruct((EP, 16), _F32),
        ],
        compiler_params=pltpu.CompilerParams(
            dimension_semantics=("parallel",)),
    )(gA, gB, xr, xc, w_d, W_m2T, b_m2r, W_c1T, b_c1r, w_c2)

    agg2, cacc2 = _sc_scatter(msg, crd, col_p, NP)

    def node_body(h_ref, xp_ref, a0_ref, a1_ref, c0_ref, c1_ref,
                  wna_ref, wnb_ref, bn1_ref, wn2_ref, bn2_ref,
                  lng_ref, lnb_ref, hn_ref, xo_ref):
        agg = a0_ref[0] + a1_ref[0]
        csum = c0_ref[0] + c1_ref[0]
        deg = csum[:, 3:4]
        xo_ref[...] = xp_ref[...] + csum / (deg + 1.0)
        pre = (_dot(h_ref[...], wna_ref[...]) + _dot(agg, wnb_ref[...])
               + bn1_ref[...])
        hn = h_ref[...] + _dot(_silu(pre), wn2_ref[...]) + bn2_ref[...]
        mu = jnp.mean(hn, axis=1, keepdims=True)
        var = jnp.mean((hn - mu) ** 2, axis=1, keepdims=True)
        hn_ref[...] = ((hn - mu) / jnp.sqrt(var + 1e-5) * lng_ref[...]
                       + lnb_ref[...])

    h_new, xo = pl.pallas_call(
        node_body,
        grid=(N // BN,),
        in_specs=[
            pl.BlockSpec((BN, H), lambda i: (i, 0)),
            pl.BlockSpec((BN, 16), lambda i: (i, 0)),
            pl.BlockSpec((1, BN, H), lambda i: (0, i, 0)),
            pl.BlockSpec((1, BN, H), lambda i: (1, i, 0)),
            pl.BlockSpec((1, BN, 16), lambda i: (0, i, 0)),
            pl.BlockSpec((1, BN, 16), lambda i: (1, i, 0)),
            pl.BlockSpec((H, H), lambda i: (0, 0)),
            pl.BlockSpec((H, H), lambda i: (0, 0)),
            pl.BlockSpec((1, H), lambda i: (0, 0)),
            pl.BlockSpec((H, H), lambda i: (0, 0)),
            pl.BlockSpec((1, H), lambda i: (0, 0)),
            pl.BlockSpec((1, H), lambda i: (0, 0)),
            pl.BlockSpec((1, H), lambda i: (0, 0)),
        ],
        out_specs=[
            pl.BlockSpec((BN, H), lambda i: (i, 0)),
            pl.BlockSpec((BN, 16), lambda i: (i, 0)),
        ],
        out_shape=[
            jax.ShapeDtypeStruct((N, H), _F32),
            jax.ShapeDtypeStruct((N, 16), _F32),
        ],
        compiler_params=pltpu.CompilerParams(
            dimension_semantics=("parallel",)),
    )(h, xpad, agg2, agg2, cacc2, cacc2,
      W_n1aT, W_n1bT, b_n1r, W_n2T, b_n2r, ln_gr, ln_br)

    return (h_new, xo[:, :3])
